# baseline (speedup 1.0000x reference)
import jax
import jax.numpy as jnp
from jax.experimental import pallas as pl

B = 8192
D = 256
K = 1024
E = 8
BETA = 1.0

BM = 512
NB = B // BM

_HI = jax.lax.Precision.HIGHEST


def _rvq_body(x_ref, cb0_ref, cb1_ref, cb2_ref, gw_ref, gb_ref,
              xq_ref, idx_ref, loss_ref):
    x = x_ref[...]
    r = x
    xq = jnp.zeros_like(x)
    idx_cols = []
    losses = []
    for cb_ref in (cb0_ref, cb1_ref, cb2_ref):
        cb = cb_ref[...]
        csum = jnp.sum(cb * cb, axis=1)
        rsum = jnp.sum(r * r, axis=1)
        m = jax.lax.dot_general(r, cb, (((1,), (1,)), ((), ())),
                                precision=jax.lax.Precision.DEFAULT)
        d = rsum[:, None] - 2.0 * m + csum[None, :]
        idx = jnp.argmin(d, axis=1).astype(jnp.int32)
        onehot = (idx[:, None] ==
                  jax.lax.broadcasted_iota(jnp.int32, (BM, K), 1)
                  ).astype(jnp.float32)
        q = jax.lax.dot_general(onehot, cb, (((1,), (0,)), ((), ())),
                                precision=_HI)
        diff = q - r
        losses.append(jnp.sum(diff * diff))
        x_res = r + diff
        r = r - x_res
        xq = xq + x_res
        idx_cols.append(idx)
    logits = jax.lax.dot_general(x, gw_ref[...], (((1,), (0,)), ((), ())),
                                 precision=jax.lax.Precision.DEFAULT) + gb_ref[...]
    probs = jax.nn.softmax(logits, axis=-1)
    expert = jnp.argmax(probs, axis=-1).astype(jnp.int32)
    idx_cols.append(expert)

    xq_ref[...] = xq
    idx_ref[...] = jnp.stack(idx_cols, axis=-1)
    loss_ref[...] = jnp.stack(losses).reshape(1, 1, 3)


def kernel(x, codebook_0, codebook_1, codebook_2, gate_W, gate_b,
           labels_0, labels_1, labels_2):
    del labels_0, labels_1, labels_2
    gate_b2 = gate_b.reshape(1, E)
    xq, idx, loss_parts = pl.pallas_call(
        _rvq_body,
        grid=(NB,),
        in_specs=[
            pl.BlockSpec((BM, D), lambda i: (i, 0)),
            pl.BlockSpec((K, D), lambda i: (0, 0)),
            pl.BlockSpec((K, D), lambda i: (0, 0)),
            pl.BlockSpec((K, D), lambda i: (0, 0)),
            pl.BlockSpec((D, E), lambda i: (0, 0)),
            pl.BlockSpec((1, E), lambda i: (0, 0)),
        ],
        out_specs=[
            pl.BlockSpec((BM, D), lambda i: (i, 0)),
            pl.BlockSpec((BM, 4), lambda i: (i, 0)),
            pl.BlockSpec((1, 1, 3), lambda i: (i, 0, 0)),
        ],
        out_shape=[
            jax.ShapeDtypeStruct((B, D), jnp.float32),
            jax.ShapeDtypeStruct((B, 4), jnp.int32),
            jax.ShapeDtypeStruct((NB, 1, 3), jnp.float32),
        ],
    )(x, codebook_0, codebook_1, codebook_2, gate_W, gate_b2)
    mean_losses = jnp.sum(loss_parts) * ((1.0 + BETA) / (3.0 * B * D))
    return (xq, mean_losses, idx)


# speedup vs baseline: 1.3520x; 1.3520x over previous
import jax
import jax.numpy as jnp
from jax.experimental import pallas as pl

B = 8192
D = 256
K = 1024
E = 8
BETA = 1.0

BM = 512
NB = B // BM

_HI = jax.lax.Precision.HIGHEST


def _rvq_body(x_ref, cb0_ref, cb1_ref, cb2_ref, gw_ref, gb_ref,
              *split_and_out_refs):
    split_refs = split_and_out_refs[:9]
    xq_ref, idx_ref, loss_ref = split_and_out_refs[9:]
    x = x_ref[...]
    r = x
    xq = jnp.zeros_like(x)
    idx_cols = []
    losses = []
    for s, cb_ref in enumerate((cb0_ref, cb1_ref, cb2_ref)):
        cb = cb_ref[...]
        csum = jnp.sum(cb * cb, axis=1)
        rsum = jnp.sum(r * r, axis=1)
        m = jax.lax.dot_general(r, cb, (((1,), (1,)), ((), ())),
                                precision=jax.lax.Precision.DEFAULT)
        d = rsum[:, None] - 2.0 * m + csum[None, :]
        idx = jnp.argmin(d, axis=1).astype(jnp.int32)
        onehot = (idx[:, None] ==
                  jax.lax.broadcasted_iota(jnp.int32, (BM, K), 1)
                  ).astype(jnp.bfloat16)
        hi, mid, lo = (split_refs[3 * s][...], split_refs[3 * s + 1][...],
                       split_refs[3 * s + 2][...])
        dims = (((1,), (0,)), ((), ()))
        q = ((jax.lax.dot_general(onehot, hi, dims,
                                  preferred_element_type=jnp.float32)
              + jax.lax.dot_general(onehot, mid, dims,
                                    preferred_element_type=jnp.float32))
             + jax.lax.dot_general(onehot, lo, dims,
                                   preferred_element_type=jnp.float32))
        diff = q - r
        losses.append(jnp.sum(diff * diff))
        x_res = r + diff
        r = r - x_res
        xq = xq + x_res
        idx_cols.append(idx)
    logits = jax.lax.dot_general(x, gw_ref[...], (((1,), (0,)), ((), ())),
                                 precision=jax.lax.Precision.DEFAULT) + gb_ref[...]
    probs = jax.nn.softmax(logits, axis=-1)
    expert = jnp.argmax(probs, axis=-1).astype(jnp.int32)
    idx_cols.append(expert)

    xq_ref[...] = xq
    idx_ref[...] = jnp.stack(idx_cols, axis=-1)
    loss_ref[...] = jnp.stack(losses).reshape(1, 1, 3)


def kernel(x, codebook_0, codebook_1, codebook_2, gate_W, gate_b,
           labels_0, labels_1, labels_2):
    del labels_0, labels_1, labels_2
    gate_b2 = gate_b.reshape(1, E)
    splits = []
    for cb in (codebook_0, codebook_1, codebook_2):
        hi = cb.astype(jnp.bfloat16)
        rem = cb - hi.astype(jnp.float32)
        mid = rem.astype(jnp.bfloat16)
        lo = (rem - mid.astype(jnp.float32)).astype(jnp.bfloat16)
        splits += [hi, mid, lo]
    xq, idx, loss_parts = pl.pallas_call(
        _rvq_body,
        grid=(NB,),
        in_specs=[
            pl.BlockSpec((BM, D), lambda i: (i, 0)),
            pl.BlockSpec((K, D), lambda i: (0, 0)),
            pl.BlockSpec((K, D), lambda i: (0, 0)),
            pl.BlockSpec((K, D), lambda i: (0, 0)),
            pl.BlockSpec((D, E), lambda i: (0, 0)),
            pl.BlockSpec((1, E), lambda i: (0, 0)),
        ] + [pl.BlockSpec((K, D), lambda i: (0, 0))] * 9,
        out_specs=[
            pl.BlockSpec((BM, D), lambda i: (i, 0)),
            pl.BlockSpec((BM, 4), lambda i: (i, 0)),
            pl.BlockSpec((1, 1, 3), lambda i: (i, 0, 0)),
        ],
        out_shape=[
            jax.ShapeDtypeStruct((B, D), jnp.float32),
            jax.ShapeDtypeStruct((B, 4), jnp.int32),
            jax.ShapeDtypeStruct((NB, 1, 3), jnp.float32),
        ],
    )(x, codebook_0, codebook_1, codebook_2, gate_W, gate_b2, *splits)
    mean_losses = jnp.sum(loss_parts) * ((1.0 + BETA) / (3.0 * B * D))
    return (xq, mean_losses, idx)
